# baseline (speedup 1.0000x reference)
import functools

import jax
import jax.numpy as jnp
from jax import lax
from jax.experimental import pallas as pl
from jax.experimental.pallas import tpu as pltpu
from jax.experimental.pallas import tpu_sc as plsc

N = 10000
E = 320000
EH = E // 2
D = 128
DP = D // 2
F_IN = 3
T = 8
OUT = 3

NC = 2
NS = 16
NW = NC * NS
NPAD = 10240
RPT = NPAD // NS


def _mesh():
    return plsc.VectorSubcoreMesh(core_axis_name="c", subcore_axis_name="s")


@functools.cache
def _sc_gather_kernel(ne):
    epw = ne // NW
    chunk = 40 if epw % 80 else 80
    nchunk = epw // chunk
    nchunk2 = nchunk + 1
    kg = 3

    def body(hs_hbm, hd_hbm, src_hbm, dst_hbm, g_out,
             si_v, di_v,
             gs0, gd0, gs1, gd1, gs2, gd2,
             sg0, sg1, sg2, so0, so1, so2):
        wid = lax.axis_index("s") * NC + lax.axis_index("c")
        base = wid * epw
        pltpu.sync_copy(src_hbm.at[pl.ds(base, epw)], si_v)
        pltpu.sync_copy(dst_hbm.at[pl.ds(base, epw)], di_v)

        gsb = (gs0, gs1, gs2)
        gdb = (gd0, gd1, gd2)
        sg = (sg0, sg1, sg2)
        so = (so0, so1, so2)

        def issue_gather(c, j):
            io = pl.multiple_of(lax.rem(c, nchunk) * chunk, 8)
            pltpu.async_copy(hs_hbm.at[si_v.at[pl.ds(io, chunk)]],
                             gsb[j], sg[j])
            pltpu.async_copy(hd_hbm.at[di_v.at[pl.ds(io, chunk)]],
                             gdb[j], sg[j])

        def wait_gather(j):
            pltpu.make_async_copy(hs_hbm.at[pl.ds(0, chunk)], gsb[j],
                                  sg[j]).wait()
            pltpu.make_async_copy(hd_hbm.at[pl.ds(0, chunk)], gdb[j],
                                  sg[j]).wait()

        def add_rows(j):
            a, b = gsb[j], gdb[j]

            def arow(r, carry):
                for g in range(D // 16):
                    sl = pl.ds(g * 16, 16)
                    a[r, sl] = a[r, sl] + b[r, sl]
                return carry

            lax.fori_loop(0, chunk, arow, 0)

        def issue_out(c, j):
            o = pl.multiple_of(base + lax.rem(c, nchunk) * chunk, 8)
            pltpu.async_copy(gsb[j], g_out.at[pl.ds(o, chunk)], so[j])

        def wait_out(j):
            pltpu.make_async_copy(gsb[j], g_out.at[pl.ds(0, chunk)],
                                  so[j]).wait()

        issue_gather(0, 0)
        issue_gather(1, 1)

        def super_body(cc, carry):
            for j in range(kg):
                c = cc * kg + j
                jj = (j + kg - 1) % kg
                if j == 0:
                    @pl.when(cc > 0)
                    def _():
                        wait_out(jj)
                        issue_gather(c + 2, jj)

                    @pl.when(cc == 0)
                    def _():
                        issue_gather(c + 2, jj)
                else:
                    wait_out(jj)

                    @pl.when(c + 2 < nchunk2)
                    def _():
                        issue_gather(c + 2, jj)
                wait_gather(j)
                add_rows(j)
                issue_out(c, j)
            return carry

        lax.fori_loop(0, nchunk2 // kg, super_body, 0)
        wait_out(kg - 1)

    buf = pltpu.VMEM((chunk, D), jnp.float32)
    return pl.kernel(
        body,
        out_type=jax.ShapeDtypeStruct((ne, D), jnp.float32),
        mesh=_mesh(),
        scratch_types=[
            pltpu.VMEM((epw,), jnp.int32),
            pltpu.VMEM((epw,), jnp.int32),
            buf, buf, buf, buf, buf, buf,
            pltpu.SemaphoreType.DMA,
            pltpu.SemaphoreType.DMA,
            pltpu.SemaphoreType.DMA,
            pltpu.SemaphoreType.DMA,
            pltpu.SemaphoreType.DMA,
            pltpu.SemaphoreType.DMA,
        ],
    )


def _sc_gather(hs, hd, src, dst):
    return _sc_gather_kernel(src.shape[0])(hs, hd, src, dst)


@functools.cache
def _sc_scatter_kernel(ne):
    epw = ne // NW
    chunk = 40 if epw % 80 else 80
    nchunk = epw // chunk
    nchunk2 = nchunk + 1

    def body(e_hbm, dst_hbm, out_hbm,
             i0, i1, e0, e1, z_v, si0, si1, se0, se1, acc_sh):
        cid = lax.axis_index("c")
        sid = lax.axis_index("s")

        def zrow(r, carry):
            def zcol(jc, carry2):
                z_v[r, pl.ds(jc * 16, 16)] = jnp.zeros((16,), jnp.float32)
                return carry2
            return lax.fori_loop(0, D // 16, zcol, carry)

        lax.fori_loop(0, 80, zrow, 0)
        rbase = pl.multiple_of(sid * RPT, 8)
        for b in range(RPT // 80):
            pltpu.sync_copy(z_v, acc_sh.at[pl.ds(rbase + b * 80, 80)])
        plsc.subcore_barrier()

        ebase = (cid * NS + sid) * epw
        ib = (i0, i1)
        eb = (e0, e1)
        sib = (si0, si1)
        seb = (se0, se1)

        def issue_loads(c, j):
            o = pl.multiple_of(ebase + lax.rem(c, nchunk) * chunk, 8)
            pltpu.async_copy(dst_hbm.at[pl.ds(o, chunk)], ib[j], sib[j])
            pltpu.async_copy(e_hbm.at[pl.ds(o, chunk)], eb[j], seb[j])

        def wait_loads(j):
            pltpu.make_async_copy(dst_hbm.at[pl.ds(0, chunk)], ib[j],
                                  sib[j]).wait()
            pltpu.make_async_copy(e_hbm.at[pl.ds(0, chunk)], eb[j],
                                  seb[j]).wait()

        issue_loads(0, 0)
        issue_loads(1, 1)

        def super_body(cc, carry):
            for j in range(2):
                c = cc * 2 + j
                wait_loads(j)
                if j == 0:
                    pltpu.sync_copy(eb[j], acc_sh.at[ib[j]], add=True)
                else:
                    @pl.when(c < nchunk)
                    def _():
                        pltpu.sync_copy(eb[j], acc_sh.at[ib[j]], add=True)

                @pl.when(c + 2 < nchunk2)
                def _():
                    issue_loads(c + 2, j)
            return carry

        lax.fori_loop(0, nchunk2 // 2, super_body, 0)
        plsc.subcore_barrier()

        for b in range(RPT // 80):
            r = pl.multiple_of(rbase + b * 80, 8)
            pltpu.sync_copy(acc_sh.at[pl.ds(r, 80)], z_v)
            pltpu.sync_copy(z_v, out_hbm.at[cid, pl.ds(r, 80)])

    return pl.kernel(
        body,
        out_type=jax.ShapeDtypeStruct((NC, NPAD, D), jnp.float32),
        mesh=_mesh(),
        scratch_types=[
            pltpu.VMEM((chunk,), jnp.int32),
            pltpu.VMEM((chunk,), jnp.int32),
            pltpu.VMEM((chunk, D), jnp.float32),
            pltpu.VMEM((chunk, D), jnp.float32),
            pltpu.VMEM((80, D), jnp.float32),
            pltpu.SemaphoreType.DMA,
            pltpu.SemaphoreType.DMA,
            pltpu.SemaphoreType.DMA,
            pltpu.SemaphoreType.DMA,
            pltpu.VMEM_SHARED((NPAD, D), jnp.float32),
        ],
    )


def _sc_scatter(e, dst):
    return _sc_scatter_kernel(dst.shape[0])(e, dst)


BN = 2000
BE = 2560
BEH = 2000


def _mlp2_kern(x_ref, w1_ref, b1_ref, w2_ref, b2_ref, o_ref):
    t = jnp.dot(x_ref[...], w1_ref[...],
                preferred_element_type=jnp.float32) + b1_ref[...]
    u = jnp.maximum(t, 0.0).astype(jnp.bfloat16)
    o_ref[...] = (jnp.dot(u, w2_ref[...], preferred_element_type=jnp.float32)
                  + b2_ref[...]).astype(o_ref.dtype)


def _mlp2_call(xb, w1, b1, w2, b2, br, out_dtype=jnp.float32):
    rows, k = xb.shape
    dout = w2.shape[1]
    return pl.pallas_call(
        _mlp2_kern,
        grid=(rows // br,),
        in_specs=[
            pl.BlockSpec((br, k), lambda i: (i, 0)),
            pl.BlockSpec((k, D), lambda i: (0, 0)),
            pl.BlockSpec((1, D), lambda i: (0, 0)),
            pl.BlockSpec((D, dout), lambda i: (0, 0)),
            pl.BlockSpec((1, dout), lambda i: (0, 0)),
        ],
        out_specs=pl.BlockSpec((br, dout), lambda i: (i, 0)),
        out_shape=jax.ShapeDtypeStruct((rows, dout), out_dtype),
    )(xb, w1, b1, w2, b2)


def _enc_node_kern(x_ref, w1_ref, b1_ref, w2_ref, b2_ref, wa_ref, wb_ref,
                   h_ref, hs_ref, hd_ref):
    t = jnp.dot(x_ref[...], w1_ref[...],
                preferred_element_type=jnp.float32) + b1_ref[...]
    u = jnp.maximum(t, 0.0).astype(jnp.bfloat16)
    h = jnp.dot(u, w2_ref[...],
                preferred_element_type=jnp.float32) + b2_ref[...]
    h_ref[...] = h
    hb = h.astype(jnp.bfloat16)
    hs_ref[...] = jnp.dot(hb, wa_ref[...], preferred_element_type=jnp.float32)
    hd_ref[...] = jnp.dot(hb, wb_ref[...], preferred_element_type=jnp.float32)


def _enc_node_call(xb, w1, b1, w2, b2, wa, wb):
    k = xb.shape[1]
    wspec = pl.BlockSpec((D, D), lambda i: (0, 0))
    nspec = pl.BlockSpec((BN, D), lambda i: (i, 0))
    return pl.pallas_call(
        _enc_node_kern,
        grid=(N // BN,),
        in_specs=[
            pl.BlockSpec((BN, k), lambda i: (i, 0)),
            pl.BlockSpec((k, D), lambda i: (0, 0)),
            pl.BlockSpec((1, D), lambda i: (0, 0)),
            wspec,
            pl.BlockSpec((1, D), lambda i: (0, 0)),
            wspec, wspec,
        ],
        out_specs=[nspec, nspec, nspec],
        out_shape=[jax.ShapeDtypeStruct((N, D), jnp.float32)] * 3,
    )(xb, w1, b1, w2, b2, wa, wb)


def _edge_upd_kern(g_ref, e_ref, wc_ref, b1_ref, w2_ref, b2_ref, o_ref):
    e = e_ref[...]
    eb = e if e.dtype == jnp.bfloat16 else e.astype(jnp.bfloat16)
    t = (g_ref[...] + b1_ref[...]
         + jnp.dot(eb, wc_ref[...], preferred_element_type=jnp.float32))
    u = jnp.maximum(t, 0.0).astype(jnp.bfloat16)
    o_ref[...] = (e.astype(jnp.float32)
                  + jnp.dot(u, w2_ref[...],
                            preferred_element_type=jnp.float32) + b2_ref[...])


def _edge_upd_call(g, e, wc, b1, w2, b2):
    ne = e.shape[0]
    br = 2560 if ne % 2560 == 0 else BEH
    return pl.pallas_call(
        _edge_upd_kern,
        grid=(ne // br,),
        in_specs=[
            pl.BlockSpec((br, D), lambda i: (i, 0)),
            pl.BlockSpec((br, D), lambda i: (i, 0)),
            pl.BlockSpec((D, D), lambda i: (0, 0)),
            pl.BlockSpec((1, D), lambda i: (0, 0)),
            pl.BlockSpec((D, D), lambda i: (0, 0)),
            pl.BlockSpec((1, D), lambda i: (0, 0)),
        ],
        out_specs=pl.BlockSpec((br, D), lambda i: (i, 0)),
        out_shape=jax.ShapeDtypeStruct((ne, D), jnp.float32),
    )(g, e, wc, b1, w2, b2)


def _node_core(h_ref, p0_ref, p1_ref, wh_ref, wa_ref, b1_ref, w2_ref,
               b2_ref):
    h = h_ref[...]
    agg = (p0_ref[...] + p1_ref[...]).astype(jnp.bfloat16)
    t = (jnp.dot(h.astype(jnp.bfloat16), wh_ref[...],
                 preferred_element_type=jnp.float32)
         + jnp.dot(agg, wa_ref[...], preferred_element_type=jnp.float32)
         + b1_ref[...])
    u = jnp.maximum(t, 0.0).astype(jnp.bfloat16)
    return h + jnp.dot(u, w2_ref[...],
                       preferred_element_type=jnp.float32) + b2_ref[...]


def _node_upd_mid_kern(h_ref, p0_ref, p1_ref, wh_ref, wa_ref, b1_ref,
                       w2_ref, b2_ref, wa2_ref, wb2_ref,
                       o_ref, hs_ref, hd_ref):
    hn = _node_core(h_ref, p0_ref, p1_ref, wh_ref, wa_ref, b1_ref, w2_ref,
                    b2_ref)
    o_ref[...] = hn
    hb = hn.astype(jnp.bfloat16)
    hs_ref[...] = jnp.dot(hb, wa2_ref[...],
                          preferred_element_type=jnp.float32)
    hd_ref[...] = jnp.dot(hb, wb2_ref[...],
                          preferred_element_type=jnp.float32)


def _node_upd_mid_call(h, p0, p1, wh, wa, b1, w2, b2, wa2, wb2):
    pspec = pl.BlockSpec((BN, D), lambda i: (i, 0))
    wspec = pl.BlockSpec((D, D), lambda i: (0, 0))
    bspec = pl.BlockSpec((1, D), lambda i: (0, 0))
    return pl.pallas_call(
        _node_upd_mid_kern,
        grid=(N // BN,),
        in_specs=[pspec, pspec, pspec, wspec, wspec, bspec, wspec, bspec,
                  wspec, wspec],
        out_specs=[pspec, pspec, pspec],
        out_shape=[jax.ShapeDtypeStruct((N, D), jnp.float32)] * 3,
    )(h, p0, p1, wh, wa, b1, w2, b2, wa2, wb2)


def _node_upd_dec_kern(h_ref, p0_ref, p1_ref, wh_ref, wa_ref, b1_ref,
                       w2_ref, b2_ref, wd1_ref, bd1_ref, wd2_ref, bd2_ref,
                       xt_ref, dt_ref, y_ref):
    hn = _node_core(h_ref, p0_ref, p1_ref, wh_ref, wa_ref, b1_ref, w2_ref,
                    b2_ref)
    t = (jnp.dot(hn.astype(jnp.bfloat16), wd1_ref[...],
                 preferred_element_type=jnp.float32) + bd1_ref[...])
    u = jnp.maximum(t, 0.0).astype(jnp.bfloat16)
    delta = jnp.dot(u, wd2_ref[...],
                    preferred_element_type=jnp.float32) + bd2_ref[...]
    y_ref[...] = xt_ref[...] + delta * dt_ref[...]


def _node_upd_dec_call(h, p0, p1, wh, wa, b1, w2, b2, wd1, bd1, wd2, bd2,
                       xt, dt):
    pspec = pl.BlockSpec((BN, D), lambda i: (i, 0))
    wspec = pl.BlockSpec((D, D), lambda i: (0, 0))
    bspec = pl.BlockSpec((1, D), lambda i: (0, 0))
    return pl.pallas_call(
        _node_upd_dec_kern,
        grid=(N // BN,),
        in_specs=[pspec, pspec, pspec, wspec, wspec, bspec, wspec, bspec,
                  wspec, bspec,
                  pl.BlockSpec((D, OUT), lambda i: (0, 0)),
                  pl.BlockSpec((1, OUT), lambda i: (0, 0)),
                  pl.BlockSpec((BN, OUT), lambda i: (i, 0)),
                  pl.BlockSpec((BN, 1), lambda i: (i, 0))],
        out_specs=pl.BlockSpec((BN, OUT), lambda i: (i, 0)),
        out_shape=jax.ShapeDtypeStruct((N, OUT), jnp.float32),
    )(h, p0, p1, wh, wa, b1, w2, b2, wd1, bd1, wd2, bd2, xt, dt)


def kernel(x, node_mass, x_initial, edge_attr, pos, edge_surf_index,
           edge_index, delta_t, params):
    p = params
    bf = jnp.bfloat16
    x_t = x[:, :, -1]
    enc_in = jnp.concatenate(
        [x.reshape(N, -1), node_mass, x_initial.reshape(N, -1)],
        axis=-1).astype(bf)
    src = edge_index[0].astype(jnp.int32)
    dst = edge_index[1].astype(jnp.int32)

    def row(b):
        return b.reshape(1, -1)

    def lw(l):
        we1 = p[f"L{l}_We1"]
        return (we1[:D].astype(bf), we1[D:2 * D].astype(bf),
                we1[2 * D:].astype(bf))

    wa0, wb0, wc0 = lw(0)
    wa1, wb1, wc1 = lw(1)

    h, hs, hd = _enc_node_call(enc_in, p["W_enc1"].astype(bf),
                               row(p["b_enc1"]), p["W_enc2"].astype(bf),
                               row(p["b_enc2"]), wa0, wb0)
    ea = edge_attr.astype(bf)
    e = _mlp2_call(ea, p["W_eenc1"].astype(bf), row(p["b_eenc1"]),
                   p["W_eenc2"].astype(bf), row(p["b_eenc2"]), 2560,
                   out_dtype=bf)

    hs, hd, e = jax.lax.optimization_barrier((hs, hd, e))

    g = _sc_gather(hs, hd, src, dst)
    e = _edge_upd_call(g, e, wc0, row(p["L0_be1"]),
                       p["L0_We2"].astype(bf), row(p["L0_be2"]))
    parts = _sc_scatter(e, dst)
    wn1 = p["L0_Wn1"]
    h, hs, hd = _node_upd_mid_call(h, parts[0, :N], parts[1, :N],
                                   wn1[:D].astype(bf), wn1[D:].astype(bf),
                                   row(p["L0_bn1"]),
                                   p["L0_Wn2"].astype(bf), row(p["L0_bn2"]),
                                   wa1, wb1)

    g = _sc_gather(hs, hd, src, dst)
    e = _edge_upd_call(g, e, wc1, row(p["L1_be1"]),
                       p["L1_We2"].astype(bf), row(p["L1_be2"]))
    parts = _sc_scatter(e, dst)
    wn1 = p["L1_Wn1"]
    y = _node_upd_dec_call(h, parts[0, :N], parts[1, :N],
                           wn1[:D].astype(bf), wn1[D:].astype(bf),
                           row(p["L1_bn1"]),
                           p["L1_Wn2"].astype(bf), row(p["L1_bn2"]),
                           p["W_dec1"].astype(bf), row(p["b_dec1"]),
                           p["W_dec2"].astype(bf), row(p["b_dec2"]),
                           x_t, delta_t[:, None])
    return y


# speedup vs baseline: 1.1657x; 1.1657x over previous
import functools

import jax
import jax.numpy as jnp
from jax import lax
from jax.experimental import pallas as pl
from jax.experimental.pallas import tpu as pltpu
from jax.experimental.pallas import tpu_sc as plsc

N = 10000
E = 320000
EH = E // 2
D = 128
DP = D // 2
F_IN = 3
T = 8
OUT = 3

NC = 2
NS = 16
NW = NC * NS
NPAD = 10240
RPT = NPAD // NS


def _mesh():
    return plsc.VectorSubcoreMesh(core_axis_name="c", subcore_axis_name="s")


@functools.cache
def _sc_gather_kernel(ne):
    epw = ne // NW
    chunk = 40 if epw % 80 else 80
    nchunk = epw // chunk
    nchunk2 = nchunk + 1
    kg = 3

    def body(hs_hbm, hd_hbm, src_hbm, dst_hbm, g_out,
             si_v, di_v,
             gs0, gd0, gs1, gd1, gs2, gd2,
             sg0, sg1, sg2, so0, so1, so2):
        wid = lax.axis_index("s") * NC + lax.axis_index("c")
        base = wid * epw
        pltpu.sync_copy(src_hbm.at[pl.ds(base, epw)], si_v)
        pltpu.sync_copy(dst_hbm.at[pl.ds(base, epw)], di_v)

        gsb = (gs0, gs1, gs2)
        gdb = (gd0, gd1, gd2)
        sg = (sg0, sg1, sg2)
        so = (so0, so1, so2)

        def issue_gather(c, j):
            io = pl.multiple_of(lax.rem(c, nchunk) * chunk, 8)
            pltpu.async_copy(hs_hbm.at[si_v.at[pl.ds(io, chunk)]],
                             gsb[j], sg[j])
            pltpu.async_copy(hd_hbm.at[di_v.at[pl.ds(io, chunk)]],
                             gdb[j], sg[j])

        def wait_gather(j):
            pltpu.make_async_copy(hs_hbm.at[pl.ds(0, chunk)], gsb[j],
                                  sg[j]).wait()
            pltpu.make_async_copy(hd_hbm.at[pl.ds(0, chunk)], gdb[j],
                                  sg[j]).wait()

        def add_rows(j):
            a, b = gsb[j], gdb[j]

            def arow(rr, carry):
                for u in range(2):
                    r = rr * 2 + u
                    for g in range(D // 16):
                        sl = pl.ds(g * 16, 16)
                        a[r, sl] = a[r, sl] + b[r, sl]
                return carry

            lax.fori_loop(0, chunk // 2, arow, 0)

        def issue_out(c, j):
            o = pl.multiple_of(base + lax.rem(c, nchunk) * chunk, 8)
            pltpu.async_copy(gsb[j], g_out.at[pl.ds(o, chunk)], so[j])

        def wait_out(j):
            pltpu.make_async_copy(gsb[j], g_out.at[pl.ds(0, chunk)],
                                  so[j]).wait()

        issue_gather(0, 0)
        issue_gather(1, 1)

        def super_body(cc, carry):
            for j in range(kg):
                c = cc * kg + j
                jj = (j + kg - 1) % kg
                if j == 0:
                    @pl.when(cc > 0)
                    def _():
                        wait_out(jj)
                        issue_gather(c + 2, jj)

                    @pl.when(cc == 0)
                    def _():
                        issue_gather(c + 2, jj)
                else:
                    wait_out(jj)

                    @pl.when(c + 2 < nchunk2)
                    def _():
                        issue_gather(c + 2, jj)
                wait_gather(j)
                add_rows(j)
                issue_out(c, j)
            return carry

        lax.fori_loop(0, nchunk2 // kg, super_body, 0)
        wait_out(kg - 1)

    buf = pltpu.VMEM((chunk, D), jnp.float32)
    return pl.kernel(
        body,
        out_type=jax.ShapeDtypeStruct((ne, D), jnp.float32),
        mesh=_mesh(),
        scratch_types=[
            pltpu.VMEM((epw,), jnp.int32),
            pltpu.VMEM((epw,), jnp.int32),
            buf, buf, buf, buf, buf, buf,
            pltpu.SemaphoreType.DMA,
            pltpu.SemaphoreType.DMA,
            pltpu.SemaphoreType.DMA,
            pltpu.SemaphoreType.DMA,
            pltpu.SemaphoreType.DMA,
            pltpu.SemaphoreType.DMA,
        ],
    )


def _sc_gather(hs, hd, src, dst):
    return _sc_gather_kernel(src.shape[0])(hs, hd, src, dst)


@functools.cache
def _sc_scatter_kernel(ne):
    epw = ne // NW
    chunk = 40 if epw % 80 else 80
    nchunk = epw // chunk
    nchunk2 = nchunk + 1

    def body(e_hbm, dst_hbm, out_hbm,
             i0, i1, e0, e1, z_v, si0, si1, se0, se1, acc_sh):
        cid = lax.axis_index("c")
        sid = lax.axis_index("s")

        def zrow(r, carry):
            def zcol(jc, carry2):
                z_v[r, pl.ds(jc * 16, 16)] = jnp.zeros((16,), jnp.float32)
                return carry2
            return lax.fori_loop(0, D // 16, zcol, carry)

        lax.fori_loop(0, 80, zrow, 0)
        rbase = pl.multiple_of(sid * RPT, 8)
        for b in range(RPT // 80):
            pltpu.sync_copy(z_v, acc_sh.at[pl.ds(rbase + b * 80, 80)])
        plsc.subcore_barrier()

        ebase = (cid * NS + sid) * epw
        ib = (i0, i1)
        eb = (e0, e1)
        sib = (si0, si1)
        seb = (se0, se1)

        def issue_loads(c, j):
            o = pl.multiple_of(ebase + lax.rem(c, nchunk) * chunk, 8)
            pltpu.async_copy(dst_hbm.at[pl.ds(o, chunk)], ib[j], sib[j])
            pltpu.async_copy(e_hbm.at[pl.ds(o, chunk)], eb[j], seb[j])

        def wait_loads(j):
            pltpu.make_async_copy(dst_hbm.at[pl.ds(0, chunk)], ib[j],
                                  sib[j]).wait()
            pltpu.make_async_copy(e_hbm.at[pl.ds(0, chunk)], eb[j],
                                  seb[j]).wait()

        issue_loads(0, 0)
        issue_loads(1, 1)

        def super_body(cc, carry):
            for j in range(2):
                c = cc * 2 + j
                wait_loads(j)
                if j == 0:
                    pltpu.sync_copy(eb[j], acc_sh.at[ib[j]], add=True)
                else:
                    @pl.when(c < nchunk)
                    def _():
                        pltpu.sync_copy(eb[j], acc_sh.at[ib[j]], add=True)

                @pl.when(c + 2 < nchunk2)
                def _():
                    issue_loads(c + 2, j)
            return carry

        lax.fori_loop(0, nchunk2 // 2, super_body, 0)
        plsc.subcore_barrier()

        for b in range(RPT // 80):
            r = pl.multiple_of(rbase + b * 80, 8)
            pltpu.sync_copy(acc_sh.at[pl.ds(r, 80)], z_v)
            pltpu.sync_copy(z_v, out_hbm.at[cid, pl.ds(r, 80)])

    return pl.kernel(
        body,
        out_type=jax.ShapeDtypeStruct((NC, NPAD, D), jnp.float32),
        mesh=_mesh(),
        scratch_types=[
            pltpu.VMEM((chunk,), jnp.int32),
            pltpu.VMEM((chunk,), jnp.int32),
            pltpu.VMEM((chunk, D), jnp.float32),
            pltpu.VMEM((chunk, D), jnp.float32),
            pltpu.VMEM((80, D), jnp.float32),
            pltpu.SemaphoreType.DMA,
            pltpu.SemaphoreType.DMA,
            pltpu.SemaphoreType.DMA,
            pltpu.SemaphoreType.DMA,
            pltpu.VMEM_SHARED((NPAD, D), jnp.float32),
        ],
    )


def _sc_scatter(e, dst):
    return _sc_scatter_kernel(dst.shape[0])(e, dst)


BN = 2000
BE = 2560
BEH = 2000


def _mlp2_kern(x_ref, w1_ref, b1_ref, w2_ref, b2_ref, o_ref):
    t = jnp.dot(x_ref[...], w1_ref[...],
                preferred_element_type=jnp.float32) + b1_ref[...]
    u = jnp.maximum(t, 0.0).astype(jnp.bfloat16)
    o_ref[...] = (jnp.dot(u, w2_ref[...], preferred_element_type=jnp.float32)
                  + b2_ref[...]).astype(o_ref.dtype)


def _mlp2_call(xb, w1, b1, w2, b2, br, out_dtype=jnp.float32):
    rows, k = xb.shape
    dout = w2.shape[1]
    return pl.pallas_call(
        _mlp2_kern,
        grid=(rows // br,),
        in_specs=[
            pl.BlockSpec((br, k), lambda i: (i, 0)),
            pl.BlockSpec((k, D), lambda i: (0, 0)),
            pl.BlockSpec((1, D), lambda i: (0, 0)),
            pl.BlockSpec((D, dout), lambda i: (0, 0)),
            pl.BlockSpec((1, dout), lambda i: (0, 0)),
        ],
        out_specs=pl.BlockSpec((br, dout), lambda i: (i, 0)),
        out_shape=jax.ShapeDtypeStruct((rows, dout), out_dtype),
    )(xb, w1, b1, w2, b2)


def _enc_node_kern(x_ref, w1_ref, b1_ref, w2_ref, b2_ref, wa_ref, wb_ref,
                   h_ref, hs_ref, hd_ref):
    t = jnp.dot(x_ref[...], w1_ref[...],
                preferred_element_type=jnp.float32) + b1_ref[...]
    u = jnp.maximum(t, 0.0).astype(jnp.bfloat16)
    h = jnp.dot(u, w2_ref[...],
                preferred_element_type=jnp.float32) + b2_ref[...]
    h_ref[...] = h
    hb = h.astype(jnp.bfloat16)
    hs_ref[...] = jnp.dot(hb, wa_ref[...], preferred_element_type=jnp.float32)
    hd_ref[...] = jnp.dot(hb, wb_ref[...], preferred_element_type=jnp.float32)


def _enc_node_call(xb, w1, b1, w2, b2, wa, wb):
    k = xb.shape[1]
    wspec = pl.BlockSpec((D, D), lambda i: (0, 0))
    nspec = pl.BlockSpec((BN, D), lambda i: (i, 0))
    return pl.pallas_call(
        _enc_node_kern,
        grid=(N // BN,),
        in_specs=[
            pl.BlockSpec((BN, k), lambda i: (i, 0)),
            pl.BlockSpec((k, D), lambda i: (0, 0)),
            pl.BlockSpec((1, D), lambda i: (0, 0)),
            wspec,
            pl.BlockSpec((1, D), lambda i: (0, 0)),
            wspec, wspec,
        ],
        out_specs=[nspec, nspec, nspec],
        out_shape=[jax.ShapeDtypeStruct((N, D), jnp.float32)] * 3,
    )(xb, w1, b1, w2, b2, wa, wb)


def _edge_upd_kern(g_ref, e_ref, wc_ref, b1_ref, w2_ref, b2_ref, o_ref):
    e = e_ref[...]
    eb = e if e.dtype == jnp.bfloat16 else e.astype(jnp.bfloat16)
    t = (g_ref[...] + b1_ref[...]
         + jnp.dot(eb, wc_ref[...], preferred_element_type=jnp.float32))
    u = jnp.maximum(t, 0.0).astype(jnp.bfloat16)
    o_ref[...] = (e.astype(jnp.float32)
                  + jnp.dot(u, w2_ref[...],
                            preferred_element_type=jnp.float32) + b2_ref[...])


def _edge_upd_call(g, e, wc, b1, w2, b2):
    ne = e.shape[0]
    br = 4000 if ne % 4000 == 0 else BEH
    return pl.pallas_call(
        _edge_upd_kern,
        grid=(ne // br,),
        in_specs=[
            pl.BlockSpec((br, D), lambda i: (i, 0)),
            pl.BlockSpec((br, D), lambda i: (i, 0)),
            pl.BlockSpec((D, D), lambda i: (0, 0)),
            pl.BlockSpec((1, D), lambda i: (0, 0)),
            pl.BlockSpec((D, D), lambda i: (0, 0)),
            pl.BlockSpec((1, D), lambda i: (0, 0)),
        ],
        out_specs=pl.BlockSpec((br, D), lambda i: (i, 0)),
        out_shape=jax.ShapeDtypeStruct((ne, D), jnp.float32),
    )(g, e, wc, b1, w2, b2)


def _node_core(h_ref, p0_ref, p1_ref, wh_ref, wa_ref, b1_ref, w2_ref,
               b2_ref):
    h = h_ref[...]
    agg = (p0_ref[...] + p1_ref[...]).astype(jnp.bfloat16)
    t = (jnp.dot(h.astype(jnp.bfloat16), wh_ref[...],
                 preferred_element_type=jnp.float32)
         + jnp.dot(agg, wa_ref[...], preferred_element_type=jnp.float32)
         + b1_ref[...])
    u = jnp.maximum(t, 0.0).astype(jnp.bfloat16)
    return h + jnp.dot(u, w2_ref[...],
                       preferred_element_type=jnp.float32) + b2_ref[...]


def _node_upd_mid_kern(h_ref, p0_ref, p1_ref, wh_ref, wa_ref, b1_ref,
                       w2_ref, b2_ref, wa2_ref, wb2_ref,
                       o_ref, hs_ref, hd_ref):
    hn = _node_core(h_ref, p0_ref, p1_ref, wh_ref, wa_ref, b1_ref, w2_ref,
                    b2_ref)
    o_ref[...] = hn
    hb = hn.astype(jnp.bfloat16)
    hs_ref[...] = jnp.dot(hb, wa2_ref[...],
                          preferred_element_type=jnp.float32)
    hd_ref[...] = jnp.dot(hb, wb2_ref[...],
                          preferred_element_type=jnp.float32)


def _node_upd_mid_call(h, p0, p1, wh, wa, b1, w2, b2, wa2, wb2):
    pspec = pl.BlockSpec((BN, D), lambda i: (i, 0))
    wspec = pl.BlockSpec((D, D), lambda i: (0, 0))
    bspec = pl.BlockSpec((1, D), lambda i: (0, 0))
    return pl.pallas_call(
        _node_upd_mid_kern,
        grid=(N // BN,),
        in_specs=[pspec, pspec, pspec, wspec, wspec, bspec, wspec, bspec,
                  wspec, wspec],
        out_specs=[pspec, pspec, pspec],
        out_shape=[jax.ShapeDtypeStruct((N, D), jnp.float32)] * 3,
    )(h, p0, p1, wh, wa, b1, w2, b2, wa2, wb2)


def _node_upd_dec_kern(h_ref, p0_ref, p1_ref, wh_ref, wa_ref, b1_ref,
                       w2_ref, b2_ref, wd1_ref, bd1_ref, wd2_ref, bd2_ref,
                       xt_ref, dt_ref, y_ref):
    hn = _node_core(h_ref, p0_ref, p1_ref, wh_ref, wa_ref, b1_ref, w2_ref,
                    b2_ref)
    t = (jnp.dot(hn.astype(jnp.bfloat16), wd1_ref[...],
                 preferred_element_type=jnp.float32) + bd1_ref[...])
    u = jnp.maximum(t, 0.0).astype(jnp.bfloat16)
    delta = jnp.dot(u, wd2_ref[...],
                    preferred_element_type=jnp.float32) + bd2_ref[...]
    y_ref[...] = xt_ref[...] + delta * dt_ref[...]


def _node_upd_dec_call(h, p0, p1, wh, wa, b1, w2, b2, wd1, bd1, wd2, bd2,
                       xt, dt):
    pspec = pl.BlockSpec((BN, D), lambda i: (i, 0))
    wspec = pl.BlockSpec((D, D), lambda i: (0, 0))
    bspec = pl.BlockSpec((1, D), lambda i: (0, 0))
    return pl.pallas_call(
        _node_upd_dec_kern,
        grid=(N // BN,),
        in_specs=[pspec, pspec, pspec, wspec, wspec, bspec, wspec, bspec,
                  wspec, bspec,
                  pl.BlockSpec((D, OUT), lambda i: (0, 0)),
                  pl.BlockSpec((1, OUT), lambda i: (0, 0)),
                  pl.BlockSpec((BN, OUT), lambda i: (i, 0)),
                  pl.BlockSpec((BN, 1), lambda i: (i, 0))],
        out_specs=pl.BlockSpec((BN, OUT), lambda i: (i, 0)),
        out_shape=jax.ShapeDtypeStruct((N, OUT), jnp.float32),
    )(h, p0, p1, wh, wa, b1, w2, b2, wd1, bd1, wd2, bd2, xt, dt)


def kernel(x, node_mass, x_initial, edge_attr, pos, edge_surf_index,
           edge_index, delta_t, params):
    p = params
    bf = jnp.bfloat16
    x_t = x[:, :, -1]
    enc_in = jnp.concatenate(
        [x.reshape(N, -1), node_mass, x_initial.reshape(N, -1)],
        axis=-1).astype(bf)
    src = edge_index[0].astype(jnp.int32)
    dst = edge_index[1].astype(jnp.int32)

    def row(b):
        return b.reshape(1, -1)

    def lw(l):
        we1 = p[f"L{l}_We1"]
        return (we1[:D].astype(bf), we1[D:2 * D].astype(bf),
                we1[2 * D:].astype(bf))

    wa0, wb0, wc0 = lw(0)
    wa1, wb1, wc1 = lw(1)

    h, hs, hd = _enc_node_call(enc_in, p["W_enc1"].astype(bf),
                               row(p["b_enc1"]), p["W_enc2"].astype(bf),
                               row(p["b_enc2"]), wa0, wb0)
    ea = edge_attr.astype(bf)
    e = _mlp2_call(ea, p["W_eenc1"].astype(bf), row(p["b_eenc1"]),
                   p["W_eenc2"].astype(bf), row(p["b_eenc2"]), 4000,
                   out_dtype=bf)

    g = _sc_gather(hs, hd, src, dst)
    e = _edge_upd_call(g, e, wc0, row(p["L0_be1"]),
                       p["L0_We2"].astype(bf), row(p["L0_be2"]))
    parts = _sc_scatter(e, dst)
    wn1 = p["L0_Wn1"]
    h, hs, hd = _node_upd_mid_call(h, parts[0, :N], parts[1, :N],
                                   wn1[:D].astype(bf), wn1[D:].astype(bf),
                                   row(p["L0_bn1"]),
                                   p["L0_Wn2"].astype(bf), row(p["L0_bn2"]),
                                   wa1, wb1)

    g = _sc_gather(hs, hd, src, dst)
    e = _edge_upd_call(g, e, wc1, row(p["L1_be1"]),
                       p["L1_We2"].astype(bf), row(p["L1_be2"]))
    parts = _sc_scatter(e, dst)
    wn1 = p["L1_Wn1"]
    y = _node_upd_dec_call(h, parts[0, :N], parts[1, :N],
                           wn1[:D].astype(bf), wn1[D:].astype(bf),
                           row(p["L1_bn1"]),
                           p["L1_Wn2"].astype(bf), row(p["L1_bn2"]),
                           p["W_dec1"].astype(bf), row(p["b_dec1"]),
                           p["W_dec2"].astype(bf), row(p["b_dec2"]),
                           x_t, delta_t[:, None])
    return y


# speedup vs baseline: 1.1903x; 1.0211x over previous
import functools

import jax
import jax.numpy as jnp
from jax import lax
from jax.experimental import pallas as pl
from jax.experimental.pallas import tpu as pltpu
from jax.experimental.pallas import tpu_sc as plsc

N = 10000
E = 320000
EH = E // 2
D = 128
DP = D // 2
F_IN = 3
T = 8
OUT = 3

NC = 2
NS = 16
NW = NC * NS
NPAD = 10240
RPT = NPAD // NS


def _mesh():
    return plsc.VectorSubcoreMesh(core_axis_name="c", subcore_axis_name="s")


@functools.cache
def _sc_gather_kernel(ne):
    epw = ne // NW
    chunk = 40 if epw % 80 else 80
    nchunk = epw // chunk
    nchunk2 = nchunk + 1
    kg = 3

    def body(hs_hbm, hd_hbm, src_hbm, dst_hbm, g_out,
             si_v, di_v,
             gs0, gd0, gs1, gd1, gs2, gd2,
             sg0, sg1, sg2, so0, so1, so2):
        wid = lax.axis_index("s") * NC + lax.axis_index("c")
        base = wid * epw
        pltpu.sync_copy(src_hbm.at[pl.ds(base, epw)], si_v)
        pltpu.sync_copy(dst_hbm.at[pl.ds(base, epw)], di_v)

        gsb = (gs0, gs1, gs2)
        gdb = (gd0, gd1, gd2)
        sg = (sg0, sg1, sg2)
        so = (so0, so1, so2)

        def issue_gather(c, j):
            io = pl.multiple_of(lax.rem(c, nchunk) * chunk, 8)
            pltpu.async_copy(hs_hbm.at[si_v.at[pl.ds(io, chunk)]],
                             gsb[j], sg[j])
            pltpu.async_copy(hd_hbm.at[di_v.at[pl.ds(io, chunk)]],
                             gdb[j], sg[j])

        def wait_gather(j):
            pltpu.make_async_copy(hs_hbm.at[pl.ds(0, chunk)], gsb[j],
                                  sg[j]).wait()
            pltpu.make_async_copy(hd_hbm.at[pl.ds(0, chunk)], gdb[j],
                                  sg[j]).wait()

        def add_rows(j):
            a, b = gsb[j], gdb[j]

            def arow(rr, carry):
                for u in range(2):
                    r = rr * 2 + u
                    for g in range(D // 16):
                        sl = pl.ds(g * 16, 16)
                        a[r, sl] = a[r, sl] + b[r, sl]
                return carry

            lax.fori_loop(0, chunk // 2, arow, 0)

        def issue_out(c, j):
            o = pl.multiple_of(base + lax.rem(c, nchunk) * chunk, 8)
            pltpu.async_copy(gsb[j], g_out.at[pl.ds(o, chunk)], so[j])

        def wait_out(j):
            pltpu.make_async_copy(gsb[j], g_out.at[pl.ds(0, chunk)],
                                  so[j]).wait()

        issue_gather(0, 0)
        issue_gather(1, 1)

        def super_body(cc, carry):
            for j in range(kg):
                c = cc * kg + j
                jj = (j + kg - 1) % kg
                if j == 0:
                    @pl.when(cc > 0)
                    def _():
                        wait_out(jj)
                        issue_gather(c + 2, jj)

                    @pl.when(cc == 0)
                    def _():
                        issue_gather(c + 2, jj)
                else:
                    wait_out(jj)

                    @pl.when(c + 2 < nchunk2)
                    def _():
                        issue_gather(c + 2, jj)
                wait_gather(j)
                add_rows(j)
                issue_out(c, j)
            return carry

        lax.fori_loop(0, nchunk2 // kg, super_body, 0)
        wait_out(kg - 1)

    buf = pltpu.VMEM((chunk, D), jnp.float32)
    return pl.kernel(
        body,
        out_type=jax.ShapeDtypeStruct((ne, D), jnp.float32),
        mesh=_mesh(),
        scratch_types=[
            pltpu.VMEM((epw,), jnp.int32),
            pltpu.VMEM((epw,), jnp.int32),
            buf, buf, buf, buf, buf, buf,
            pltpu.SemaphoreType.DMA,
            pltpu.SemaphoreType.DMA,
            pltpu.SemaphoreType.DMA,
            pltpu.SemaphoreType.DMA,
            pltpu.SemaphoreType.DMA,
            pltpu.SemaphoreType.DMA,
        ],
    )


def _sc_gather(hs, hd, src, dst):
    return _sc_gather_kernel(src.shape[0])(hs, hd, src, dst)


@functools.cache
def _sc_scatter_kernel(ne):
    epw = ne // NW
    chunk = 40 if epw % 80 else 80
    nchunk = epw // chunk
    nchunk2 = nchunk + 1

    def body(e_hbm, dst_hbm, out_hbm,
             i0, i1, e0, e1, z_v, si0, si1, se0, se1, acc_sh):
        cid = lax.axis_index("c")
        sid = lax.axis_index("s")

        def zrow(r, carry):
            def zcol(jc, carry2):
                z_v[r, pl.ds(jc * 16, 16)] = jnp.zeros((16,), jnp.float32)
                return carry2
            return lax.fori_loop(0, D // 16, zcol, carry)

        lax.fori_loop(0, 80, zrow, 0)
        rbase = pl.multiple_of(sid * RPT, 8)
        for b in range(RPT // 80):
            pltpu.sync_copy(z_v, acc_sh.at[pl.ds(rbase + b * 80, 80)])
        plsc.subcore_barrier()

        ebase = (cid * NS + sid) * epw
        ib = (i0, i1)
        eb = (e0, e1)
        sib = (si0, si1)
        seb = (se0, se1)

        def issue_loads(c, j):
            o = pl.multiple_of(ebase + lax.rem(c, nchunk) * chunk, 8)
            pltpu.async_copy(dst_hbm.at[pl.ds(o, chunk)], ib[j], sib[j])
            pltpu.async_copy(e_hbm.at[pl.ds(o, chunk)], eb[j], seb[j])

        def wait_loads(j):
            pltpu.make_async_copy(dst_hbm.at[pl.ds(0, chunk)], ib[j],
                                  sib[j]).wait()
            pltpu.make_async_copy(e_hbm.at[pl.ds(0, chunk)], eb[j],
                                  seb[j]).wait()

        issue_loads(0, 0)
        issue_loads(1, 1)

        def super_body(cc, carry):
            for j in range(2):
                c = cc * 2 + j
                wait_loads(j)
                if j == 0:
                    pltpu.sync_copy(eb[j], acc_sh.at[ib[j]], add=True)
                else:
                    @pl.when(c < nchunk)
                    def _():
                        pltpu.sync_copy(eb[j], acc_sh.at[ib[j]], add=True)

                @pl.when(c + 2 < nchunk2)
                def _():
                    issue_loads(c + 2, j)
            return carry

        lax.fori_loop(0, nchunk2 // 2, super_body, 0)
        plsc.subcore_barrier()

        for b in range(RPT // 80):
            r = pl.multiple_of(rbase + b * 80, 8)
            pltpu.sync_copy(acc_sh.at[pl.ds(r, 80)], z_v)
            pltpu.sync_copy(z_v, out_hbm.at[cid, pl.ds(r, 80)])

    return pl.kernel(
        body,
        out_type=jax.ShapeDtypeStruct((NC, NPAD, D), jnp.float32),
        mesh=_mesh(),
        scratch_types=[
            pltpu.VMEM((chunk,), jnp.int32),
            pltpu.VMEM((chunk,), jnp.int32),
            pltpu.VMEM((chunk, D), jnp.float32),
            pltpu.VMEM((chunk, D), jnp.float32),
            pltpu.VMEM((80, D), jnp.float32),
            pltpu.SemaphoreType.DMA,
            pltpu.SemaphoreType.DMA,
            pltpu.SemaphoreType.DMA,
            pltpu.SemaphoreType.DMA,
            pltpu.VMEM_SHARED((NPAD, D), jnp.float32),
        ],
    )


def _sc_scatter(e, dst):
    return _sc_scatter_kernel(dst.shape[0])(e, dst)


BN = 2000
BE = 2560
BEH = 2000


def _mlp2_kern(x_ref, w1_ref, b1_ref, w2_ref, b2_ref, o_ref):
    t = jnp.dot(x_ref[...], w1_ref[...],
                preferred_element_type=jnp.float32) + b1_ref[...]
    u = jnp.maximum(t, 0.0).astype(jnp.bfloat16)
    o_ref[...] = (jnp.dot(u, w2_ref[...], preferred_element_type=jnp.float32)
                  + b2_ref[...]).astype(o_ref.dtype)


def _mlp2_call(xb, w1, b1, w2, b2, br, out_dtype=jnp.float32):
    rows, k = xb.shape
    dout = w2.shape[1]
    return pl.pallas_call(
        _mlp2_kern,
        grid=(rows // br,),
        in_specs=[
            pl.BlockSpec((br, k), lambda i: (i, 0)),
            pl.BlockSpec((k, D), lambda i: (0, 0)),
            pl.BlockSpec((1, D), lambda i: (0, 0)),
            pl.BlockSpec((D, dout), lambda i: (0, 0)),
            pl.BlockSpec((1, dout), lambda i: (0, 0)),
        ],
        out_specs=pl.BlockSpec((br, dout), lambda i: (i, 0)),
        out_shape=jax.ShapeDtypeStruct((rows, dout), out_dtype),
    )(xb, w1, b1, w2, b2)


def _enc_node_kern(x_ref, w1_ref, b1_ref, w2_ref, b2_ref, wa_ref, wb_ref,
                   h_ref, hs_ref, hd_ref):
    t = jnp.dot(x_ref[...], w1_ref[...],
                preferred_element_type=jnp.float32) + b1_ref[...]
    u = jnp.maximum(t, 0.0).astype(jnp.bfloat16)
    h = jnp.dot(u, w2_ref[...],
                preferred_element_type=jnp.float32) + b2_ref[...]
    h_ref[...] = h
    hb = h.astype(jnp.bfloat16)
    hs_ref[...] = jnp.dot(hb, wa_ref[...], preferred_element_type=jnp.float32)
    hd_ref[...] = jnp.dot(hb, wb_ref[...], preferred_element_type=jnp.float32)


def _enc_node_call(xb, w1, b1, w2, b2, wa, wb):
    k = xb.shape[1]
    wspec = pl.BlockSpec((D, D), lambda i: (0, 0))
    nspec = pl.BlockSpec((BN, D), lambda i: (i, 0))
    return pl.pallas_call(
        _enc_node_kern,
        grid=(N // BN,),
        in_specs=[
            pl.BlockSpec((BN, k), lambda i: (i, 0)),
            pl.BlockSpec((k, D), lambda i: (0, 0)),
            pl.BlockSpec((1, D), lambda i: (0, 0)),
            wspec,
            pl.BlockSpec((1, D), lambda i: (0, 0)),
            wspec, wspec,
        ],
        out_specs=[nspec, nspec, nspec],
        out_shape=[jax.ShapeDtypeStruct((N, D), jnp.float32)] * 3,
    )(xb, w1, b1, w2, b2, wa, wb)


def _edge_upd_kern(g_ref, e_ref, wc_ref, b1_ref, w2_ref, b2_ref, o_ref):
    e = e_ref[...]
    eb = e if e.dtype == jnp.bfloat16 else e.astype(jnp.bfloat16)
    t = (g_ref[...] + b1_ref[...]
         + jnp.dot(eb, wc_ref[...], preferred_element_type=jnp.float32))
    u = jnp.maximum(t, 0.0).astype(jnp.bfloat16)
    o_ref[...] = (e.astype(jnp.float32)
                  + jnp.dot(u, w2_ref[...],
                            preferred_element_type=jnp.float32) + b2_ref[...])


def _edge_upd_call(g, e, wc, b1, w2, b2):
    ne = e.shape[0]
    br = 8000 if ne % 8000 == 0 else BEH
    return pl.pallas_call(
        _edge_upd_kern,
        grid=(ne // br,),
        in_specs=[
            pl.BlockSpec((br, D), lambda i: (i, 0)),
            pl.BlockSpec((br, D), lambda i: (i, 0)),
            pl.BlockSpec((D, D), lambda i: (0, 0)),
            pl.BlockSpec((1, D), lambda i: (0, 0)),
            pl.BlockSpec((D, D), lambda i: (0, 0)),
            pl.BlockSpec((1, D), lambda i: (0, 0)),
        ],
        out_specs=pl.BlockSpec((br, D), lambda i: (i, 0)),
        out_shape=jax.ShapeDtypeStruct((ne, D), jnp.float32),
    )(g, e, wc, b1, w2, b2)


def _node_core(h_ref, p0_ref, p1_ref, wh_ref, wa_ref, b1_ref, w2_ref,
               b2_ref):
    h = h_ref[...]
    agg = (p0_ref[...] + p1_ref[...]).astype(jnp.bfloat16)
    t = (jnp.dot(h.astype(jnp.bfloat16), wh_ref[...],
                 preferred_element_type=jnp.float32)
         + jnp.dot(agg, wa_ref[...], preferred_element_type=jnp.float32)
         + b1_ref[...])
    u = jnp.maximum(t, 0.0).astype(jnp.bfloat16)
    return h + jnp.dot(u, w2_ref[...],
                       preferred_element_type=jnp.float32) + b2_ref[...]


def _node_upd_mid_kern(h_ref, p0_ref, p1_ref, wh_ref, wa_ref, b1_ref,
                       w2_ref, b2_ref, wa2_ref, wb2_ref,
                       o_ref, hs_ref, hd_ref):
    hn = _node_core(h_ref, p0_ref, p1_ref, wh_ref, wa_ref, b1_ref, w2_ref,
                    b2_ref)
    o_ref[...] = hn
    hb = hn.astype(jnp.bfloat16)
    hs_ref[...] = jnp.dot(hb, wa2_ref[...],
                          preferred_element_type=jnp.float32)
    hd_ref[...] = jnp.dot(hb, wb2_ref[...],
                          preferred_element_type=jnp.float32)


def _node_upd_mid_call(h, p0, p1, wh, wa, b1, w2, b2, wa2, wb2):
    pspec = pl.BlockSpec((BN, D), lambda i: (i, 0))
    wspec = pl.BlockSpec((D, D), lambda i: (0, 0))
    bspec = pl.BlockSpec((1, D), lambda i: (0, 0))
    return pl.pallas_call(
        _node_upd_mid_kern,
        grid=(N // BN,),
        in_specs=[pspec, pspec, pspec, wspec, wspec, bspec, wspec, bspec,
                  wspec, wspec],
        out_specs=[pspec, pspec, pspec],
        out_shape=[jax.ShapeDtypeStruct((N, D), jnp.float32)] * 3,
    )(h, p0, p1, wh, wa, b1, w2, b2, wa2, wb2)


def _node_upd_dec_kern(h_ref, p0_ref, p1_ref, wh_ref, wa_ref, b1_ref,
                       w2_ref, b2_ref, wd1_ref, bd1_ref, wd2_ref, bd2_ref,
                       xt_ref, dt_ref, y_ref):
    hn = _node_core(h_ref, p0_ref, p1_ref, wh_ref, wa_ref, b1_ref, w2_ref,
                    b2_ref)
    t = (jnp.dot(hn.astype(jnp.bfloat16), wd1_ref[...],
                 preferred_element_type=jnp.float32) + bd1_ref[...])
    u = jnp.maximum(t, 0.0).astype(jnp.bfloat16)
    delta = jnp.dot(u, wd2_ref[...],
                    preferred_element_type=jnp.float32) + bd2_ref[...]
    y_ref[...] = xt_ref[...] + delta * dt_ref[...]


def _node_upd_dec_call(h, p0, p1, wh, wa, b1, w2, b2, wd1, bd1, wd2, bd2,
                       xt, dt):
    pspec = pl.BlockSpec((BN, D), lambda i: (i, 0))
    wspec = pl.BlockSpec((D, D), lambda i: (0, 0))
    bspec = pl.BlockSpec((1, D), lambda i: (0, 0))
    return pl.pallas_call(
        _node_upd_dec_kern,
        grid=(N // BN,),
        in_specs=[pspec, pspec, pspec, wspec, wspec, bspec, wspec, bspec,
                  wspec, bspec,
                  pl.BlockSpec((D, OUT), lambda i: (0, 0)),
                  pl.BlockSpec((1, OUT), lambda i: (0, 0)),
                  pl.BlockSpec((BN, OUT), lambda i: (i, 0)),
                  pl.BlockSpec((BN, 1), lambda i: (i, 0))],
        out_specs=pl.BlockSpec((BN, OUT), lambda i: (i, 0)),
        out_shape=jax.ShapeDtypeStruct((N, OUT), jnp.float32),
    )(h, p0, p1, wh, wa, b1, w2, b2, wd1, bd1, wd2, bd2, xt, dt)


def kernel(x, node_mass, x_initial, edge_attr, pos, edge_surf_index,
           edge_index, delta_t, params):
    p = params
    bf = jnp.bfloat16
    x_t = x[:, :, -1]
    enc_in = jnp.concatenate(
        [x.reshape(N, -1), node_mass, x_initial.reshape(N, -1)],
        axis=-1).astype(bf)
    src = edge_index[0].astype(jnp.int32)
    dst = edge_index[1].astype(jnp.int32)

    def row(b):
        return b.reshape(1, -1)

    def lw(l):
        we1 = p[f"L{l}_We1"]
        return (we1[:D].astype(bf), we1[D:2 * D].astype(bf),
                we1[2 * D:].astype(bf))

    wa0, wb0, wc0 = lw(0)
    wa1, wb1, wc1 = lw(1)

    h, hs, hd = _enc_node_call(enc_in, p["W_enc1"].astype(bf),
                               row(p["b_enc1"]), p["W_enc2"].astype(bf),
                               row(p["b_enc2"]), wa0, wb0)
    ea = edge_attr.astype(bf)
    e = _mlp2_call(ea, p["W_eenc1"].astype(bf), row(p["b_eenc1"]),
                   p["W_eenc2"].astype(bf), row(p["b_eenc2"]), 8000,
                   out_dtype=bf)

    g = _sc_gather(hs, hd, src, dst)
    e = _edge_upd_call(g, e, wc0, row(p["L0_be1"]),
                       p["L0_We2"].astype(bf), row(p["L0_be2"]))
    parts = _sc_scatter(e, dst)
    wn1 = p["L0_Wn1"]
    h, hs, hd = _node_upd_mid_call(h, parts[0, :N], parts[1, :N],
                                   wn1[:D].astype(bf), wn1[D:].astype(bf),
                                   row(p["L0_bn1"]),
                                   p["L0_Wn2"].astype(bf), row(p["L0_bn2"]),
                                   wa1, wb1)

    g = _sc_gather(hs, hd, src, dst)
    e = _edge_upd_call(g, e, wc1, row(p["L1_be1"]),
                       p["L1_We2"].astype(bf), row(p["L1_be2"]))
    parts = _sc_scatter(e, dst)
    wn1 = p["L1_Wn1"]
    y = _node_upd_dec_call(h, parts[0, :N], parts[1, :N],
                           wn1[:D].astype(bf), wn1[D:].astype(bf),
                           row(p["L1_bn1"]),
                           p["L1_Wn2"].astype(bf), row(p["L1_bn2"]),
                           p["W_dec1"].astype(bf), row(p["b_dec1"]),
                           p["W_dec2"].astype(bf), row(p["b_dec2"]),
                           x_t, delta_t[:, None])
    return y
